# baseline (speedup 1.0000x reference)
import jax
import jax.numpy as jnp
from jax import lax
from jax.experimental import pallas as pl
from jax.experimental.pallas import tpu as pltpu
from jax.experimental.pallas import tpu_sc as plsc

N = 10000
NP = 10112
E = 320000
D_IN = 128
D_HID = 256
D_OUT = 128

NC, NS = 2, 16
NW = NC * NS
EB = 128
HB = 64
RPT = 80
R2 = NW * RPT
E_PAD = R2 * EB
DUMMY = N
CHK = 4
NSB = 2 * CHK
ROWS_OUT = NP // NS
ZB = 8


def _make_sc_agg(with_deg: bool):
    out_type = [jax.ShapeDtypeStruct((NC, NP, 128), jnp.float32)]
    if with_deg:
        out_type.append(jax.ShapeDtypeStruct((NC, NP, 128), jnp.float32))
    scratch = [
        pltpu.VMEM_SHARED((NP, 128), jnp.float32),
        pltpu.VMEM((2 * CHK, EB), jnp.int32),
        pltpu.VMEM((NSB, HB), jnp.int32),
        pltpu.VMEM((NSB, HB), jnp.int32),
        pltpu.VMEM((2 * HB, 128), jnp.float32),
        pltpu.VMEM((ZB, 128), jnp.float32),
        pltpu.SemaphoreType.DMA,
        pltpu.SemaphoreType.DMA,
        pltpu.SemaphoreType.DMA,
        pltpu.SemaphoreType.DMA,
    ]
    if with_deg:
        scratch += [
            pltpu.VMEM_SHARED((NP + 128,), jnp.float32),
            pltpu.VMEM((EB,), jnp.float32),
            pltpu.VMEM((EB,), jnp.float32),
            pltpu.VMEM((ROWS_OUT + 16,), jnp.float32),
            pltpu.VMEM((8, 128), jnp.float32),
            pltpu.SemaphoreType.DMA,
            pltpu.SemaphoreType.DMA,
        ]

    def body(feat, src_h, dst_h, *rest):
        if with_deg:
            (acc_out, deg_out, acc_sh, idx_v, srcx, dstx, rows_v, zeros_v,
             sg0, sg1, ss0, ss1,
             deg_sh, ones1d, zeros1d, buf1d, exp_v, sem_d, sem_e) = rest
        else:
            (acc_out, acc_sh, idx_v, srcx, dstx, rows_v, zeros_v,
             sg0, sg1, ss0, ss1) = rest
        sem_g = (sg0, sg1)
        sem_s = (ss0, ss1)
        c = lax.axis_index("c")
        s = lax.axis_index("s")
        wid = s * NC + c
        t0 = s * ROWS_OUT

        zf = jnp.zeros((16,), jnp.float32)
        for r in range(ZB):
            for k in range(128 // 16):
                zeros_v[r, pl.ds(k * 16, 16)] = zf

        def zloop(k, carry):
            pltpu.sync_copy(zeros_v, acc_sh.at[pl.ds(t0 + k * ZB, ZB)])
            return carry
        lax.fori_loop(0, ROWS_OUT // ZB, zloop, 0)

        if with_deg:
            of = jnp.full((16,), 1.0, jnp.float32)
            for g in range(EB // 16):
                ones1d[pl.ds(16 * g, 16)] = of
                zeros1d[pl.ds(16 * g, 16)] = zf

            def zdeg(k, carry):
                pltpu.sync_copy(zeros1d, deg_sh.at[pl.ds(t0 + k * EB, EB)])
                return carry
            lax.fori_loop(0, (ROWS_OUT + 128) // EB, zdeg, 0)
        plsc.subcore_barrier()

        def buf(b):
            return rows_v.at[pl.ds(HB * b, HB)]

        def chunk_loop(cc, carry):
            base = wid * RPT + cc * CHK
            pltpu.sync_copy(src_h.at[pl.ds(base, CHK)], idx_v.at[pl.ds(0, CHK)])
            pltpu.sync_copy(dst_h.at[pl.ds(base, CHK)], idx_v.at[pl.ds(CHK, CHK)])
            for j in range(CHK):
                for h in range(2):
                    for g in range(HB // 16):
                        srcx[2 * j + h, pl.ds(16 * g, 16)] = (
                            idx_v[j, pl.ds(HB * h + 16 * g, 16)])
                        dstx[2 * j + h, pl.ds(16 * g, 16)] = (
                            idx_v[CHK + j, pl.ds(HB * h + 16 * g, 16)])
            dg = [None, None]
            dsc = [None, None]
            dd = []
            for i in range(NSB):
                b = i & 1
                if dsc[b] is not None:
                    dsc[b].wait()
                dg[b] = pltpu.async_copy(feat.at[srcx.at[i]], buf(b),
                                         sem_g[b])
                if with_deg and (i & 1) == 0:
                    dd.append(pltpu.async_copy(
                        ones1d, deg_sh.at[idx_v.at[CHK + i // 2]], sem_d,
                        add=True))
                if i >= 1:
                    o = (i - 1) & 1
                    dg[o].wait()
                    dsc[o] = pltpu.async_copy(buf(o),
                                              acc_sh.at[dstx.at[i - 1]],
                                              sem_s[o], add=True)
            dg[(NSB - 1) & 1].wait()
            dsc[(NSB - 1) & 1] = pltpu.async_copy(
                buf((NSB - 1) & 1), acc_sh.at[dstx.at[NSB - 1]],
                sem_s[(NSB - 1) & 1], add=True)
            dsc[0].wait()
            dsc[1].wait()
            for d in dd:
                d.wait()
            return carry

        lax.fori_loop(0, RPT // CHK, chunk_loop, 0)
        plsc.subcore_barrier()
        pltpu.sync_copy(acc_sh.at[pl.ds(t0, ROWS_OUT)],
                        acc_out.at[c, pl.ds(t0, ROWS_OUT)])

        if with_deg:
            pltpu.sync_copy(deg_sh.at[pl.ds(t0, ROWS_OUT + 16)], buf1d)

            def exp_sync(k, carry):
                for r in range(8):
                    exp_v[r, pl.ds(0, 16)] = buf1d[pl.ds(k * 8 + r, 16)]
                pltpu.async_copy(
                    exp_v, deg_out.at[c, pl.ds(t0 + k * 8, 8)],
                    sem_e).wait()
                return carry

            lax.fori_loop(0, ROWS_OUT // 8, exp_sync, 0)

    mesh = plsc.VectorSubcoreMesh(core_axis_name="c", subcore_axis_name="s",
                                  num_cores=NC, num_subcores=NS)
    return pl.kernel(body, out_type=tuple(out_type), mesh=mesh,
                     scratch_types=tuple(scratch))


_sc_agg_deg = _make_sc_agg(True)
_sc_agg = _make_sc_agg(False)

BN = NP // 8


def _tc1_body(x_ref, a0, a1, d0, d1, ws1, wn1, b1, wn2, h_ref, hw2_ref):
    deg = d0[:, 0:1] + d1[:, 0:1]
    deginv = 1.0 / jnp.maximum(deg, 1.0)
    agg = (a0[...] + a1[...]) * deginv
    h = jnp.dot(x_ref[...], ws1[...], preferred_element_type=jnp.float32)
    h += jnp.dot(agg, wn1[...], preferred_element_type=jnp.float32)
    h = jnp.maximum(h + b1[...], 0.0)
    h_ref[...] = h
    hw2_ref[...] = jnp.dot(h, wn2[...], preferred_element_type=jnp.float32)


def _tc2_body(h_ref, a0, a1, d0, d1, ws2, b2, out_ref):
    deg = d0[:, 0:1] + d1[:, 0:1]
    deginv = 1.0 / jnp.maximum(deg, 1.0)
    out = jnp.dot(h_ref[...], ws2[...], preferred_element_type=jnp.float32)
    out_ref[...] = out + (a0[...] + a1[...]) * deginv + b2[...]


def _row_block(d):
    return pl.BlockSpec((BN, d), lambda i: (i, 0))


def _full_block(r, c):
    return pl.BlockSpec((r, c), lambda i: (0, 0))


_tc1 = pl.pallas_call(
    _tc1_body,
    grid=(NP // BN,),
    in_specs=[
        _row_block(D_IN), _row_block(D_IN), _row_block(D_IN),
        _row_block(128), _row_block(128),
        _full_block(D_IN, D_HID), _full_block(D_IN, D_HID),
        _full_block(1, D_HID), _full_block(D_HID, D_OUT),
    ],
    out_specs=[_row_block(D_HID), _row_block(D_OUT)],
    out_shape=[
        jax.ShapeDtypeStruct((NP, D_HID), jnp.float32),
        jax.ShapeDtypeStruct((NP, D_OUT), jnp.float32),
    ],
)

_tc2 = pl.pallas_call(
    _tc2_body,
    grid=(NP // BN,),
    in_specs=[
        _row_block(D_HID), _row_block(D_OUT), _row_block(D_OUT),
        _row_block(128), _row_block(128),
        _full_block(D_HID, D_OUT), _full_block(1, D_OUT),
    ],
    out_specs=_row_block(D_OUT),
    out_shape=jax.ShapeDtypeStruct((NP, D_OUT), jnp.float32),
)


def kernel(x, edge_index, W_self1, W_neigh1, b1, W_self2, W_neigh2, b2):
    src = edge_index[0].astype(jnp.int32)
    dst = edge_index[1].astype(jnp.int32)
    pad = E_PAD - E
    src2d = jnp.concatenate([src, jnp.zeros((pad,), jnp.int32)]).reshape(R2, EB)
    dst2d = jnp.concatenate([dst, jnp.full((pad,), DUMMY, jnp.int32)]).reshape(R2, EB)
    x_p = jnp.concatenate([x, jnp.zeros((NP - N, D_IN), jnp.float32)])

    acc1, degf = _sc_agg_deg(x_p, src2d, dst2d)
    h, hw2 = _tc1(x_p, acc1[0], acc1[1], degf[0], degf[1],
                  W_self1, W_neigh1, b1.reshape(1, D_HID), W_neigh2)
    (acc2,) = _sc_agg(hw2, src2d, dst2d)
    out = _tc2(h, acc2[0], acc2[1], degf[0], degf[1],
               W_self2, b2.reshape(1, D_OUT))
    return out[:N]


# speedup vs baseline: 1.0828x; 1.0828x over previous
import jax
import jax.numpy as jnp
from jax import lax
from jax.experimental import pallas as pl
from jax.experimental.pallas import tpu as pltpu
from jax.experimental.pallas import tpu_sc as plsc

N = 10000
NP = 10112
E = 320000
D_IN = 128
D_HID = 256
D_OUT = 128

NC, NS = 2, 16
NW = NC * NS
EB = 128
HB = 64
RPT = 80
R2 = NW * RPT
E_PAD = R2 * EB
DUMMY = N
CHK = 4
NSB = 2 * CHK
ROWS_OUT = NP // NS
ZB = 8


def _make_sc_agg(with_deg: bool):
    out_type = [jax.ShapeDtypeStruct((NC, NP, 128), jnp.float32)]
    if with_deg:
        out_type.append(jax.ShapeDtypeStruct((NC, NP, 128), jnp.float32))
    scratch = [
        pltpu.VMEM_SHARED((NP, 128), jnp.float32),
        pltpu.VMEM((2, 2 * CHK, EB), jnp.int32),
        pltpu.VMEM((2, NSB, HB), jnp.int32),
        pltpu.VMEM((2, NSB, HB), jnp.int32),
        pltpu.VMEM((2 * HB, 128), jnp.float32),
        pltpu.VMEM((ZB, 128), jnp.float32),
        pltpu.SemaphoreType.DMA,
        pltpu.SemaphoreType.DMA,
        pltpu.SemaphoreType.DMA,
        pltpu.SemaphoreType.DMA,
        pltpu.SemaphoreType.DMA,
        pltpu.SemaphoreType.DMA,
    ]
    if with_deg:
        scratch += [
            pltpu.VMEM_SHARED((NP + 128,), jnp.float32),
            pltpu.VMEM((EB,), jnp.float32),
            pltpu.VMEM((EB,), jnp.float32),
            pltpu.VMEM((ROWS_OUT + 16,), jnp.float32),
            pltpu.VMEM((8, 128), jnp.float32),
            pltpu.SemaphoreType.DMA,
            pltpu.SemaphoreType.DMA,
        ]

    def body(feat, src_h, dst_h, *rest):
        if with_deg:
            (acc_out, deg_out, acc_sh, idxb, srcx, dstx, rows_v, zeros_v,
             sg0, sg1, ss0, ss1, si0, si1,
             deg_sh, ones1d, zeros1d, buf1d, exp_v, sem_d, sem_e) = rest
        else:
            (acc_out, acc_sh, idxb, srcx, dstx, rows_v, zeros_v,
             sg0, sg1, ss0, ss1, si0, si1) = rest
        sem_g = (sg0, sg1)
        sem_s = (ss0, ss1)
        c = lax.axis_index("c")
        s = lax.axis_index("s")
        wid = s * NC + c
        t0 = s * ROWS_OUT

        zf = jnp.zeros((16,), jnp.float32)
        for r in range(ZB):
            for k in range(128 // 16):
                zeros_v[r, pl.ds(k * 16, 16)] = zf

        def zloop(k, carry):
            pltpu.sync_copy(zeros_v, acc_sh.at[pl.ds(t0 + k * ZB, ZB)])
            return carry
        lax.fori_loop(0, ROWS_OUT // ZB, zloop, 0)

        if with_deg:
            of = jnp.full((16,), 1.0, jnp.float32)
            for g in range(EB // 16):
                ones1d[pl.ds(16 * g, 16)] = of
                zeros1d[pl.ds(16 * g, 16)] = zf

            def zdeg(k, carry):
                pltpu.sync_copy(zeros1d, deg_sh.at[pl.ds(t0 + k * EB, EB)])
                return carry
            lax.fori_loop(0, (ROWS_OUT + 128) // EB, zdeg, 0)
        plsc.subcore_barrier()

        def buf(b):
            return rows_v.at[pl.ds(HB * b, HB)]

        NCH = RPT // CHK
        sem_i = (si0, si1)

        def stage(cc):
            p = cc % 2
            base = wid * RPT + cc * CHK
            d1 = pltpu.async_copy(src_h.at[pl.ds(base, CHK)],
                                  idxb.at[p, pl.ds(0, CHK)], sem_i[p])
            d2 = pltpu.async_copy(dst_h.at[pl.ds(base, CHK)],
                                  idxb.at[p, pl.ds(CHK, CHK)], sem_i[p])
            return (d1, d2)

        dstage = {0: stage(0)}
        dd_chunks = {}
        dg = [None, None]
        dsc = [None, None]
        for cc in range(NCH):
            p = cc % 2
            for d in dstage.pop(cc):
                d.wait()
            if with_deg and (cc - 1) in dd_chunks:
                for d in dd_chunks.pop(cc - 1):
                    d.wait()
            if cc + 1 < NCH:
                dstage[cc + 1] = stage(cc + 1)
            for j in range(CHK):
                for h in range(2):
                    for g in range(HB // 16):
                        srcx[p, 2 * j + h, pl.ds(16 * g, 16)] = (
                            idxb[p, j, pl.ds(HB * h + 16 * g, 16)])
                        dstx[p, 2 * j + h, pl.ds(16 * g, 16)] = (
                            idxb[p, CHK + j, pl.ds(HB * h + 16 * g, 16)])
            ddl = []
            for i in range(NSB):
                gi = cc * NSB + i
                b = gi & 1
                if dsc[b] is not None:
                    dsc[b].wait()
                dg[b] = pltpu.async_copy(feat.at[srcx.at[p, i]], buf(b),
                                         sem_g[b])
                if with_deg and (i & 1) == 0:
                    ddl.append(pltpu.async_copy(
                        ones1d, deg_sh.at[idxb.at[p, CHK + i // 2]], sem_d,
                        add=True))
                if gi >= 1:
                    o = (gi - 1) & 1
                    dg[o].wait()
                    pi, ii = divmod(gi - 1, NSB)
                    dsc[o] = pltpu.async_copy(
                        buf(o), acc_sh.at[dstx.at[pi % 2, ii]],
                        sem_s[o], add=True)
            if with_deg:
                dd_chunks[cc] = ddl
        gl = NCH * NSB - 1
        o = gl & 1
        dg[o].wait()
        pi, ii = divmod(gl, NSB)
        dsc[o] = pltpu.async_copy(buf(o), acc_sh.at[dstx.at[pi % 2, ii]],
                                  sem_s[o], add=True)
        dsc[0].wait()
        dsc[1].wait()
        if with_deg:
            for ddl in dd_chunks.values():
                for d in ddl:
                    d.wait()
        plsc.subcore_barrier()
        pltpu.sync_copy(acc_sh.at[pl.ds(t0, ROWS_OUT)],
                        acc_out.at[c, pl.ds(t0, ROWS_OUT)])

        if with_deg:
            pltpu.sync_copy(deg_sh.at[pl.ds(t0, ROWS_OUT + 16)], buf1d)

            def exp_sync(k, carry):
                for r in range(8):
                    exp_v[r, pl.ds(0, 16)] = buf1d[pl.ds(k * 8 + r, 16)]
                pltpu.async_copy(
                    exp_v, deg_out.at[c, pl.ds(t0 + k * 8, 8)],
                    sem_e).wait()
                return carry

            lax.fori_loop(0, ROWS_OUT // 8, exp_sync, 0)

    mesh = plsc.VectorSubcoreMesh(core_axis_name="c", subcore_axis_name="s",
                                  num_cores=NC, num_subcores=NS)
    return pl.kernel(body, out_type=tuple(out_type), mesh=mesh,
                     scratch_types=tuple(scratch))


_sc_agg_deg = _make_sc_agg(True)
_sc_agg = _make_sc_agg(False)

BN = NP // 8


def _tc1_body(x_ref, a0, a1, d0, d1, ws1, wn1, b1, wn2, h_ref, hw2_ref):
    deg = d0[:, 0:1] + d1[:, 0:1]
    deginv = 1.0 / jnp.maximum(deg, 1.0)
    agg = (a0[...] + a1[...]) * deginv
    h = jnp.dot(x_ref[...], ws1[...], preferred_element_type=jnp.float32)
    h += jnp.dot(agg, wn1[...], preferred_element_type=jnp.float32)
    h = jnp.maximum(h + b1[...], 0.0)
    h_ref[...] = h
    hw2_ref[...] = jnp.dot(h, wn2[...], preferred_element_type=jnp.float32)


def _tc2_body(h_ref, a0, a1, d0, d1, ws2, b2, out_ref):
    deg = d0[:, 0:1] + d1[:, 0:1]
    deginv = 1.0 / jnp.maximum(deg, 1.0)
    out = jnp.dot(h_ref[...], ws2[...], preferred_element_type=jnp.float32)
    out_ref[...] = out + (a0[...] + a1[...]) * deginv + b2[...]


def _row_block(d):
    return pl.BlockSpec((BN, d), lambda i: (i, 0))


def _full_block(r, c):
    return pl.BlockSpec((r, c), lambda i: (0, 0))


_tc1 = pl.pallas_call(
    _tc1_body,
    grid=(NP // BN,),
    in_specs=[
        _row_block(D_IN), _row_block(D_IN), _row_block(D_IN),
        _row_block(128), _row_block(128),
        _full_block(D_IN, D_HID), _full_block(D_IN, D_HID),
        _full_block(1, D_HID), _full_block(D_HID, D_OUT),
    ],
    out_specs=[_row_block(D_HID), _row_block(D_OUT)],
    out_shape=[
        jax.ShapeDtypeStruct((NP, D_HID), jnp.float32),
        jax.ShapeDtypeStruct((NP, D_OUT), jnp.float32),
    ],
)

_tc2 = pl.pallas_call(
    _tc2_body,
    grid=(NP // BN,),
    in_specs=[
        _row_block(D_HID), _row_block(D_OUT), _row_block(D_OUT),
        _row_block(128), _row_block(128),
        _full_block(D_HID, D_OUT), _full_block(1, D_OUT),
    ],
    out_specs=_row_block(D_OUT),
    out_shape=jax.ShapeDtypeStruct((NP, D_OUT), jnp.float32),
)


def kernel(x, edge_index, W_self1, W_neigh1, b1, W_self2, W_neigh2, b2):
    src = edge_index[0].astype(jnp.int32)
    dst = edge_index[1].astype(jnp.int32)
    pad = E_PAD - E
    src2d = jnp.concatenate([src, jnp.zeros((pad,), jnp.int32)]).reshape(R2, EB)
    dst2d = jnp.concatenate([dst, jnp.full((pad,), DUMMY, jnp.int32)]).reshape(R2, EB)
    x_p = jnp.concatenate([x, jnp.zeros((NP - N, D_IN), jnp.float32)])

    acc1, degf = _sc_agg_deg(x_p, src2d, dst2d)
    h, hw2 = _tc1(x_p, acc1[0], acc1[1], degf[0], degf[1],
                  W_self1, W_neigh1, b1.reshape(1, D_HID), W_neigh2)
    (acc2,) = _sc_agg(hw2, src2d, dst2d)
    out = _tc2(h, acc2[0], acc2[1], degf[0], degf[1],
               W_self2, b2.reshape(1, D_OUT))
    return out[:N]
